# baseline (speedup 1.0000x reference)
import functools

import jax
import jax.numpy as jnp
from jax import lax
from jax.experimental import pallas as pl
from jax.experimental.pallas import tpu as pltpu
from jax.experimental.pallas import tpu_sc as plsc

N = 10000
E = 160000
IN_C = 64
OUT_C = 32
EDGE_DIM = 16
HIDDEN = 64
AUG = 40

NC = 2
NS = 16
NW = NC * NS
CH = 128
E_PAD = 163840
EPW = E_PAD // NW
NCHUNK = EPW // CH
N_ACC = 10016
RPZ = N_ACC // NS


NPIPE = 4
ECH = E_PAD // NPIPE
CPW = ECH // NW // CH


XW = 128
GB = CPW // 2


def _sc_gather_chunk(x128, src2, ci):
    mesh = plsc.VectorSubcoreMesh(core_axis_name="c", subcore_axis_name="s")

    @functools.partial(
        pl.kernel,
        mesh=mesh,
        out_type=jax.ShapeDtypeStruct((ECH, XW), jnp.float32),
        compiler_params=pltpu.CompilerParams(use_tc_tiling_on_sc=False),
        scratch_types=[
            pltpu.VMEM((CPW, CH), jnp.int32),
            pltpu.VMEM((GB * CH, XW), jnp.float32),
            pltpu.SemaphoreType.DMA,
        ],
    )
    def k(x_hbm, src2_hbm, out_hbm, idx2, rows_v, sem):
        c = lax.axis_index("c")
        s = lax.axis_index("s")
        wid = s * NC + c
        pltpu.sync_copy(
            src2_hbm.at[pl.ds(ci * (ECH // CH) + wid * CPW, CPW)], idx2)
        for half in range(CPW // GB):
            for j in range(GB):
                pltpu.async_copy(x_hbm.at[idx2.at[half * GB + j]],
                                 rows_v.at[pl.ds(j * CH, CH)], sem)
            out_slice = out_hbm.at[
                pl.ds(wid * CPW * CH + half * GB * CH, GB * CH)]
            pltpu.make_async_copy(out_slice, rows_v, sem).wait()
            pltpu.sync_copy(rows_v, out_slice)

    return k(x128, src2)


def _sc_scatter(msgs_pair, dst2, zrows, cis):
    mesh = plsc.VectorSubcoreMesh(core_axis_name="c", subcore_axis_name="s")

    @functools.partial(
        pl.kernel,
        mesh=mesh,
        out_type=jax.ShapeDtypeStruct((2, N_ACC, AUG), jnp.float32),
        compiler_params=pltpu.CompilerParams(use_tc_tiling_on_sc=False),
        scratch_types=[
            pltpu.VMEM((CPW, CH), jnp.int32),
            pltpu.VMEM((CPW * CH, AUG), jnp.float32),
            pltpu.VMEM_SHARED((N_ACC, AUG), jnp.float32),
        ],
    )
    def k(m0, m1, dst_hbm, z_hbm, out_hbm, idx_v, rows_v, acc_sh):
        c = lax.axis_index("c")
        s = lax.axis_index("s")
        pltpu.sync_copy(z_hbm.at[pl.ds(s * RPZ, RPZ)],
                        acc_sh.at[pl.ds(s * RPZ, RPZ)])
        plsc.subcore_barrier()

        wid = s * NC + c
        for ci, m_hbm in zip(cis, (m0, m1)):
            pltpu.sync_copy(
                dst_hbm.at[pl.ds(ci * (ECH // CH) + wid * CPW, CPW)], idx_v)
            pltpu.sync_copy(
                m_hbm.at[pl.ds(wid * CPW * CH, CPW * CH)], rows_v)
            for j in range(CPW):
                pltpu.sync_copy(rows_v.at[pl.ds(j * CH, CH)],
                                acc_sh.at[idx_v.at[j]], add=True)
        plsc.subcore_barrier()
        pltpu.sync_copy(acc_sh.at[pl.ds(s * RPZ, RPZ)],
                        out_hbm.at[c, pl.ds(s * RPZ, RPZ)])

    return k(*msgs_pair, dst2, zrows)


BE = 640


def _edge_body(ea_ref, xj_ref, W1_ref, b1_ref, W2_ref, b2_ref, R_ref, S_ref,
               o_ref):
    h = jnp.dot(ea_ref[...], W1_ref[...], preferred_element_type=jnp.float32)
    h = h + b1_ref[...]
    h = h * jax.nn.sigmoid(h)
    q = jnp.dot(h, W2_ref[...], preferred_element_type=jnp.float32)
    q = q + b2_ref[...]
    xje = jnp.dot(xj_ref[:, :IN_C], R_ref[...],
                  preferred_element_type=jnp.float32)
    p = q * xje
    p = p[:, :1024] + p[:, 1024:]
    p = p[:, :512] + p[:, 512:]
    p = p[:, :256] + p[:, 256:]
    p = p[:, :128] + p[:, 128:]
    acc = jnp.dot(p, S_ref[...], preferred_element_type=jnp.float32)
    o_ref[:, 0:OUT_C] = acc
    o_ref[:, OUT_C:OUT_C + 1] = jnp.ones((BE, 1), jnp.float32)
    o_ref[:, OUT_C + 1:AUG] = jnp.zeros((BE, AUG - OUT_C - 1), jnp.float32)


def _tc_edge(ea, xj, W1, b1, W2, b2, ci):
    grid = (ECH // BE,)
    off = ci * (ECH // BE)
    last = E // BE - 1
    R = jnp.kron(jnp.eye(IN_C, dtype=jnp.float32),
                 jnp.ones((1, OUT_C), jnp.float32))
    S = jnp.tile(jnp.eye(OUT_C, dtype=jnp.float32), (4, 1))
    return pl.pallas_call(
        _edge_body,
        grid=grid,
        in_specs=[
            pl.BlockSpec(
                (BE, EDGE_DIM),
                lambda i, off=off, last=last: (jnp.minimum(i + off, last), 0)),
            pl.BlockSpec((BE, XW), lambda i: (i, 0)),
            pl.BlockSpec((EDGE_DIM, HIDDEN), lambda i: (0, 0)),
            pl.BlockSpec((1, HIDDEN), lambda i: (0, 0)),
            pl.BlockSpec((HIDDEN, IN_C * OUT_C), lambda i: (0, 0)),
            pl.BlockSpec((1, IN_C * OUT_C), lambda i: (0, 0)),
            pl.BlockSpec((IN_C, IN_C * OUT_C), lambda i: (0, 0)),
            pl.BlockSpec((4 * OUT_C, OUT_C), lambda i: (0, 0)),
        ],
        out_specs=pl.BlockSpec((BE, AUG), lambda i: (i, 0)),
        out_shape=jax.ShapeDtypeStruct((ECH, AUG), jnp.float32),
    )(ea, xj, W1, b1.reshape(1, HIDDEN), W2,
      b2.reshape(1, IN_C * OUT_C), R, S)


BN = 2000


def _node_body(p_ref, q_ref, x_ref, root_ref, bias_ref, g_ref, beta_ref,
               Wres_ref, bres_ref, o_ref):
    p = (p_ref[0] + p_ref[1]) + (q_ref[0] + q_ref[1])
    summed = p[:, 0:OUT_C]
    cnt = p[:, OUT_C:OUT_C + 1]
    aggr = summed / jnp.maximum(cnt, 1.0)
    xb = x_ref[...]
    out = aggr + jnp.dot(xb, root_ref[...],
                         preferred_element_type=jnp.float32) + bias_ref[...]
    mu = jnp.mean(out, axis=1, keepdims=True)
    var = jnp.mean((out - mu) * (out - mu), axis=1, keepdims=True)
    out = (out - mu) * lax.rsqrt(var + 1e-5) * g_ref[...] + beta_ref[...]
    res = jnp.dot(xb, Wres_ref[...],
                  preferred_element_type=jnp.float32) + bres_ref[...]
    t = out + res
    o_ref[...] = t * jax.nn.sigmoid(t)


def _tc_node(pa, pb, x, root, bias, ln_gamma, ln_beta, Wres, bres):
    grid = (N // BN,)
    return pl.pallas_call(
        _node_body,
        grid=grid,
        in_specs=[
            pl.BlockSpec((2, BN, AUG), lambda i: (0, i, 0)),
            pl.BlockSpec((2, BN, AUG), lambda i: (0, i, 0)),
            pl.BlockSpec((BN, IN_C), lambda i: (i, 0)),
            pl.BlockSpec((IN_C, OUT_C), lambda i: (0, 0)),
            pl.BlockSpec((1, OUT_C), lambda i: (0, 0)),
            pl.BlockSpec((1, OUT_C), lambda i: (0, 0)),
            pl.BlockSpec((1, OUT_C), lambda i: (0, 0)),
            pl.BlockSpec((IN_C, OUT_C), lambda i: (0, 0)),
            pl.BlockSpec((1, OUT_C), lambda i: (0, 0)),
        ],
        out_specs=pl.BlockSpec((BN, OUT_C), lambda i: (i, 0)),
        out_shape=jax.ShapeDtypeStruct((N, OUT_C), jnp.float32),
    )(pa, pb, x, root, bias.reshape(1, OUT_C), ln_gamma.reshape(1, OUT_C),
      ln_beta.reshape(1, OUT_C), Wres, bres.reshape(1, OUT_C))


def kernel(x, edge_attr, W1, b1, W2, b2, root, bias, ln_gamma, ln_beta,
           Wres, bres, edge_index):
    pad = E_PAD - E
    ei2 = jnp.pad(edge_index, ((0, 0), (0, pad)),
                  constant_values=0).reshape(2, E_PAD // CH, CH)
    src2 = ei2[0]
    dst2 = jnp.where(
        jnp.arange(E_PAD).reshape(E_PAD // CH, CH) < E, ei2[1], N)
    x128 = jnp.pad(x, ((0, 0), (0, XW - IN_C)))
    zrows = jnp.zeros((N_ACC, AUG), jnp.float32)

    msgs = []
    for ci in range(NPIPE):
        xj_c = _sc_gather_chunk(x128, src2, ci)
        msgs.append(_tc_edge(edge_attr, xj_c, W1, b1, W2, b2, ci))
    pa = _sc_scatter(msgs[0:2], dst2, zrows, (0, 1))
    pb = _sc_scatter(msgs[2:4], dst2, zrows, (2, 3))
    return _tc_node(pa, pb, x, root, bias, ln_gamma, ln_beta, Wres, bres)


# speedup vs baseline: 1.2320x; 1.2320x over previous
import functools

import jax
import jax.numpy as jnp
from jax import lax
from jax.experimental import pallas as pl
from jax.experimental.pallas import tpu as pltpu
from jax.experimental.pallas import tpu_sc as plsc

N = 10000
E = 160000
IN_C = 64
OUT_C = 32
EDGE_DIM = 16
HIDDEN = 64
AUG = 40

NC = 2
NS = 16
NW = NC * NS
CH = 128
E_PAD = 163840
EPW = E_PAD // NW
NCHUNK = EPW // CH
N_ACC = 10016
RPZ = N_ACC // NS


NPIPE = 4
ECH = E_PAD // NPIPE
CPW = ECH // NW // CH


XW = 128
GB = CPW // 2


def _sc_gather_chunk(x128, src2, ci):
    mesh = plsc.VectorSubcoreMesh(core_axis_name="c", subcore_axis_name="s")

    @functools.partial(
        pl.kernel,
        mesh=mesh,
        out_type=jax.ShapeDtypeStruct((ECH, XW), jnp.float32),
        compiler_params=pltpu.CompilerParams(use_tc_tiling_on_sc=False),
        scratch_types=[
            pltpu.VMEM((CPW, CH), jnp.int32),
            pltpu.VMEM((GB * CH, XW), jnp.float32),
            pltpu.SemaphoreType.DMA,
        ],
    )
    def k(x_hbm, src2_hbm, out_hbm, idx2, rows_v, sem):
        c = lax.axis_index("c")
        s = lax.axis_index("s")
        wid = s * NC + c
        pltpu.sync_copy(
            src2_hbm.at[pl.ds(ci * (ECH // CH) + wid * CPW, CPW)], idx2)
        for half in range(CPW // GB):
            for j in range(GB):
                pltpu.async_copy(x_hbm.at[idx2.at[half * GB + j]],
                                 rows_v.at[pl.ds(j * CH, CH)], sem)
            out_slice = out_hbm.at[
                pl.ds(wid * CPW * CH + half * GB * CH, GB * CH)]
            pltpu.make_async_copy(out_slice, rows_v, sem).wait()
            pltpu.sync_copy(rows_v, out_slice)

    return k(x128, src2)


def _sc_scatter(msgs_pair, dst2, zrows, cis):
    mesh = plsc.VectorSubcoreMesh(core_axis_name="c", subcore_axis_name="s")

    @functools.partial(
        pl.kernel,
        mesh=mesh,
        out_type=jax.ShapeDtypeStruct((2, N_ACC, AUG), jnp.float32),
        compiler_params=pltpu.CompilerParams(use_tc_tiling_on_sc=False),
        scratch_types=[
            pltpu.VMEM((CPW, CH), jnp.int32),
            pltpu.VMEM((CPW * CH, AUG), jnp.float32),
            pltpu.VMEM_SHARED((N_ACC, AUG), jnp.float32),
        ],
    )
    def k(m0, m1, dst_hbm, z_hbm, out_hbm, idx_v, rows_v, acc_sh):
        c = lax.axis_index("c")
        s = lax.axis_index("s")
        pltpu.sync_copy(z_hbm.at[pl.ds(s * RPZ, RPZ)],
                        acc_sh.at[pl.ds(s * RPZ, RPZ)])
        plsc.subcore_barrier()

        wid = s * NC + c
        for ci, m_hbm in zip(cis, (m0, m1)):
            pltpu.sync_copy(
                dst_hbm.at[pl.ds(ci * (ECH // CH) + wid * CPW, CPW)], idx_v)
            pltpu.sync_copy(
                m_hbm.at[pl.ds(wid * CPW * CH, CPW * CH)], rows_v)
            for j in range(CPW):
                pltpu.sync_copy(rows_v.at[pl.ds(j * CH, CH)],
                                acc_sh.at[idx_v.at[j]], add=True)
        plsc.subcore_barrier()
        pltpu.sync_copy(acc_sh.at[pl.ds(s * RPZ, RPZ)],
                        out_hbm.at[c, pl.ds(s * RPZ, RPZ)])

    return k(*msgs_pair, dst2, zrows)


BE = 640


def _edge_body(ea_ref, xj_ref, W1_ref, b1_ref, W2_ref, b2_ref, R_ref, S_ref,
               o_ref):
    h = jnp.dot(ea_ref[...], W1_ref[...], preferred_element_type=jnp.float32)
    h = h + b1_ref[...]
    h = h * jax.nn.sigmoid(h)
    q = jnp.dot(h, W2_ref[...], preferred_element_type=jnp.float32)
    q = q + b2_ref[...]
    xje = jnp.dot(xj_ref[:, :IN_C], R_ref[...],
                  preferred_element_type=jnp.float32)
    p = q * xje
    p = p[:, :1024] + p[:, 1024:]
    p = p[:, :512] + p[:, 512:]
    p = p[:, :256] + p[:, 256:]
    p = p[:, :128] + p[:, 128:]
    acc = jnp.dot(p, S_ref[...], preferred_element_type=jnp.float32)
    o_ref[:, 0:OUT_C] = acc
    o_ref[:, OUT_C:OUT_C + 1] = jnp.ones((BE, 1), jnp.float32)
    o_ref[:, OUT_C + 1:AUG] = jnp.zeros((BE, AUG - OUT_C - 1), jnp.float32)


def _tc_edge(ea, xj, W1, b1, W2, b2, ci):
    grid = (ECH // BE,)
    off = ci * (ECH // BE)
    last = E // BE - 1
    R = jnp.kron(jnp.eye(IN_C, dtype=jnp.float32),
                 jnp.ones((1, OUT_C), jnp.float32))
    S = jnp.tile(jnp.eye(OUT_C, dtype=jnp.float32), (4, 1))
    return pl.pallas_call(
        _edge_body,
        grid=grid,
        in_specs=[
            pl.BlockSpec(
                (BE, EDGE_DIM),
                lambda i, off=off, last=last: (jnp.minimum(i + off, last), 0)),
            pl.BlockSpec((BE, XW), lambda i: (i, 0)),
            pl.BlockSpec((EDGE_DIM, HIDDEN), lambda i: (0, 0)),
            pl.BlockSpec((1, HIDDEN), lambda i: (0, 0)),
            pl.BlockSpec((HIDDEN, IN_C * OUT_C), lambda i: (0, 0)),
            pl.BlockSpec((1, IN_C * OUT_C), lambda i: (0, 0)),
            pl.BlockSpec((IN_C, IN_C * OUT_C), lambda i: (0, 0)),
            pl.BlockSpec((4 * OUT_C, OUT_C), lambda i: (0, 0)),
        ],
        out_specs=pl.BlockSpec((BE, AUG), lambda i: (i, 0)),
        out_shape=jax.ShapeDtypeStruct((ECH, AUG), jnp.float32),
    )(ea, xj, W1, b1.reshape(1, HIDDEN), W2,
      b2.reshape(1, IN_C * OUT_C), R, S)


def _pack_x_body(x_ref, o_ref):
    o_ref[:, :IN_C] = x_ref[...]
    o_ref[:, IN_C:] = jnp.zeros((N, XW - IN_C), jnp.float32)


def _tc_pack_x(x):
    return pl.pallas_call(
        _pack_x_body,
        out_shape=jax.ShapeDtypeStruct((N, XW), jnp.float32),
    )(x)


BN = 2000


def _node_body(p_ref, q_ref, x_ref, root_ref, bias_ref, g_ref, beta_ref,
               Wres_ref, bres_ref, o_ref):
    p = (p_ref[0] + p_ref[1]) + (q_ref[0] + q_ref[1])
    summed = p[:, 0:OUT_C]
    cnt = p[:, OUT_C:OUT_C + 1]
    aggr = summed / jnp.maximum(cnt, 1.0)
    xb = x_ref[...]
    out = aggr + jnp.dot(xb, root_ref[...],
                         preferred_element_type=jnp.float32) + bias_ref[...]
    mu = jnp.mean(out, axis=1, keepdims=True)
    var = jnp.mean((out - mu) * (out - mu), axis=1, keepdims=True)
    out = (out - mu) * lax.rsqrt(var + 1e-5) * g_ref[...] + beta_ref[...]
    res = jnp.dot(xb, Wres_ref[...],
                  preferred_element_type=jnp.float32) + bres_ref[...]
    t = out + res
    o_ref[...] = t * jax.nn.sigmoid(t)


def _tc_node(pa, pb, x, root, bias, ln_gamma, ln_beta, Wres, bres):
    grid = (N // BN,)
    return pl.pallas_call(
        _node_body,
        grid=grid,
        in_specs=[
            pl.BlockSpec((2, BN, AUG), lambda i: (0, i, 0)),
            pl.BlockSpec((2, BN, AUG), lambda i: (0, i, 0)),
            pl.BlockSpec((BN, IN_C), lambda i: (i, 0)),
            pl.BlockSpec((IN_C, OUT_C), lambda i: (0, 0)),
            pl.BlockSpec((1, OUT_C), lambda i: (0, 0)),
            pl.BlockSpec((1, OUT_C), lambda i: (0, 0)),
            pl.BlockSpec((1, OUT_C), lambda i: (0, 0)),
            pl.BlockSpec((IN_C, OUT_C), lambda i: (0, 0)),
            pl.BlockSpec((1, OUT_C), lambda i: (0, 0)),
        ],
        out_specs=pl.BlockSpec((BN, OUT_C), lambda i: (i, 0)),
        out_shape=jax.ShapeDtypeStruct((N, OUT_C), jnp.float32),
    )(pa, pb, x, root, bias.reshape(1, OUT_C), ln_gamma.reshape(1, OUT_C),
      ln_beta.reshape(1, OUT_C), Wres, bres.reshape(1, OUT_C))


def kernel(x, edge_attr, W1, b1, W2, b2, root, bias, ln_gamma, ln_beta,
           Wres, bres, edge_index):
    pad = E_PAD - E
    ei2 = jnp.pad(edge_index, ((0, 0), (0, pad)),
                  constant_values=0).reshape(2, E_PAD // CH, CH)
    slot = jnp.arange(E_PAD).reshape(E_PAD // CH, CH)
    src2 = jnp.where(slot < E, ei2[0], slot % N)
    dst2 = jnp.where(slot < E, ei2[1], N)
    x128 = _tc_pack_x(x)
    zrows = jnp.zeros((N_ACC, AUG), jnp.float32)

    msgs = []
    for ci in range(NPIPE):
        xj_c = _sc_gather_chunk(x128, src2, ci)
        msgs.append(_tc_edge(edge_attr, xj_c, W1, b1, W2, b2, ci))
    pa = _sc_scatter(msgs[0:2], dst2, zrows, (0, 1))
    pb = _sc_scatter(msgs[2:4], dst2, zrows, (2, 3))
    return _tc_node(pa, pb, x, root, bias, ln_gamma, ln_beta, Wres, bres)


# speedup vs baseline: 1.2386x; 1.0054x over previous
import functools

import jax
import jax.numpy as jnp
from jax import lax
from jax.experimental import pallas as pl
from jax.experimental.pallas import tpu as pltpu
from jax.experimental.pallas import tpu_sc as plsc

N = 10000
E = 160000
IN_C = 64
OUT_C = 32
EDGE_DIM = 16
HIDDEN = 64
AUG = 32
CNT_W = 8

NC = 2
NS = 16
NW = NC * NS
CH = 128
E_PAD = 163840
EPW = E_PAD // NW
NCHUNK = EPW // CH
N_ACC = 10016
RPZ = N_ACC // NS


NPIPE = 4
ECH = E_PAD // NPIPE
CPW = ECH // NW // CH


XW = 128
GB = CPW // 2


def _sc_gather_chunk(x128, src2, ci):
    mesh = plsc.VectorSubcoreMesh(core_axis_name="c", subcore_axis_name="s")

    @functools.partial(
        pl.kernel,
        mesh=mesh,
        out_type=jax.ShapeDtypeStruct((ECH, XW), jnp.float32),
        compiler_params=pltpu.CompilerParams(use_tc_tiling_on_sc=False),
        scratch_types=[
            pltpu.VMEM((CPW, CH), jnp.int32),
            pltpu.VMEM((GB * CH, XW), jnp.float32),
            pltpu.SemaphoreType.DMA,
        ],
    )
    def k(x_hbm, src2_hbm, out_hbm, idx2, rows_v, sem):
        c = lax.axis_index("c")
        s = lax.axis_index("s")
        wid = s * NC + c
        pltpu.sync_copy(
            src2_hbm.at[pl.ds(ci * (ECH // CH) + wid * CPW, CPW)], idx2)
        for half in range(CPW // GB):
            for j in range(GB):
                pltpu.async_copy(x_hbm.at[idx2.at[half * GB + j]],
                                 rows_v.at[pl.ds(j * CH, CH)], sem)
            out_slice = out_hbm.at[
                pl.ds(wid * CPW * CH + half * GB * CH, GB * CH)]
            pltpu.make_async_copy(out_slice, rows_v, sem).wait()
            pltpu.sync_copy(rows_v, out_slice)

    return k(x128, src2)


def _sc_scatter(msgs_pair, dst2, zrows, zcnt, ones, cis):
    mesh = plsc.VectorSubcoreMesh(core_axis_name="c", subcore_axis_name="s")

    @functools.partial(
        pl.kernel,
        mesh=mesh,
        out_type=(jax.ShapeDtypeStruct((2, N_ACC, AUG), jnp.float32),
                  jax.ShapeDtypeStruct((2, N_ACC, CNT_W), jnp.float32)),
        compiler_params=pltpu.CompilerParams(use_tc_tiling_on_sc=False),
        scratch_types=[
            pltpu.VMEM((CPW, CH), jnp.int32),
            pltpu.VMEM((CPW * CH, AUG), jnp.float32),
            pltpu.VMEM((CH, CNT_W), jnp.float32),
            pltpu.VMEM_SHARED((N_ACC, AUG), jnp.float32),
            pltpu.VMEM_SHARED((N_ACC, CNT_W), jnp.float32),
        ],
    )
    def k(m0, m1, dst_hbm, z_hbm, zc_hbm, ones_hbm, out_hbm, cnt_hbm,
          idx_v, rows_v, ones_v, acc_sh, cnt_sh):
        c = lax.axis_index("c")
        s = lax.axis_index("s")
        pltpu.sync_copy(z_hbm.at[pl.ds(s * RPZ, RPZ)],
                        acc_sh.at[pl.ds(s * RPZ, RPZ)])
        pltpu.sync_copy(zc_hbm.at[pl.ds(s * RPZ, RPZ)],
                        cnt_sh.at[pl.ds(s * RPZ, RPZ)])
        pltpu.sync_copy(ones_hbm, ones_v)
        plsc.subcore_barrier()

        wid = s * NC + c
        for ci, m_hbm in zip(cis, (m0, m1)):
            pltpu.sync_copy(
                dst_hbm.at[pl.ds(ci * (ECH // CH) + wid * CPW, CPW)], idx_v)
            pltpu.sync_copy(
                m_hbm.at[pl.ds(wid * CPW * CH, CPW * CH)], rows_v)
            for j in range(CPW):
                pltpu.sync_copy(rows_v.at[pl.ds(j * CH, CH)],
                                acc_sh.at[idx_v.at[j]], add=True)
                pltpu.sync_copy(ones_v, cnt_sh.at[idx_v.at[j]], add=True)
        plsc.subcore_barrier()
        pltpu.sync_copy(acc_sh.at[pl.ds(s * RPZ, RPZ)],
                        out_hbm.at[c, pl.ds(s * RPZ, RPZ)])
        pltpu.sync_copy(cnt_sh.at[pl.ds(s * RPZ, RPZ)],
                        cnt_hbm.at[c, pl.ds(s * RPZ, RPZ)])

    return k(*msgs_pair, dst2, zrows, zcnt, ones)


BE = 640


def _edge_body(ea_ref, xj_ref, W1_ref, b1_ref, W2_ref, b2_ref, R_ref, S_ref,
               o_ref):
    h = jnp.dot(ea_ref[...], W1_ref[...], preferred_element_type=jnp.float32)
    h = h + b1_ref[...]
    h = h * jax.nn.sigmoid(h)
    q = jnp.dot(h, W2_ref[...], preferred_element_type=jnp.float32)
    q = q + b2_ref[...]
    xje = jnp.dot(xj_ref[:, :IN_C], R_ref[...],
                  preferred_element_type=jnp.float32)
    p = q * xje
    p = p[:, :1024] + p[:, 1024:]
    p = p[:, :512] + p[:, 512:]
    p = p[:, :256] + p[:, 256:]
    p = p[:, :128] + p[:, 128:]
    acc = jnp.dot(p, S_ref[...], preferred_element_type=jnp.float32)
    qn = BE // 4
    o_ref[...] = jnp.concatenate(
        [acc[0:qn], acc[qn:2 * qn], acc[2 * qn:3 * qn], acc[3 * qn:4 * qn]],
        axis=1)


def _tc_edge(ea, xj, W1, b1, W2, b2, ci):
    grid = (ECH // BE,)
    off = ci * (ECH // BE)
    last = E // BE - 1
    R = jnp.kron(jnp.eye(IN_C, dtype=jnp.float32),
                 jnp.ones((1, OUT_C), jnp.float32))
    S = jnp.tile(jnp.eye(OUT_C, dtype=jnp.float32), (4, 1))
    return pl.pallas_call(
        _edge_body,
        grid=grid,
        in_specs=[
            pl.BlockSpec(
                (BE, EDGE_DIM),
                lambda i, off=off, last=last: (jnp.minimum(i + off, last), 0)),
            pl.BlockSpec((BE, XW), lambda i: (i, 0)),
            pl.BlockSpec((EDGE_DIM, HIDDEN), lambda i: (0, 0)),
            pl.BlockSpec((1, HIDDEN), lambda i: (0, 0)),
            pl.BlockSpec((HIDDEN, IN_C * OUT_C), lambda i: (0, 0)),
            pl.BlockSpec((1, IN_C * OUT_C), lambda i: (0, 0)),
            pl.BlockSpec((IN_C, IN_C * OUT_C), lambda i: (0, 0)),
            pl.BlockSpec((4 * OUT_C, OUT_C), lambda i: (0, 0)),
        ],
        out_specs=pl.BlockSpec((BE // 4, 128), lambda i: (i, 0)),
        out_shape=jax.ShapeDtypeStruct((ECH // 4, 128), jnp.float32),
    )(ea, xj, W1, b1.reshape(1, HIDDEN), W2,
      b2.reshape(1, IN_C * OUT_C), R, S)


def _pack_x_body(x_ref, o_ref):
    o_ref[:, :IN_C] = x_ref[...]
    o_ref[:, IN_C:] = jnp.zeros((N, XW - IN_C), jnp.float32)


def _tc_pack_x(x):
    return pl.pallas_call(
        _pack_x_body,
        out_shape=jax.ShapeDtypeStruct((N, XW), jnp.float32),
    )(x)


BN = 2000


def _node_body(p_ref, q_ref, ca_ref, cb_ref, x_ref, root_ref, bias_ref,
               g_ref, beta_ref, Wres_ref, bres_ref, o_ref):
    summed = (p_ref[0] + p_ref[1]) + (q_ref[0] + q_ref[1])
    cntrow = (ca_ref[0] + ca_ref[1]) + (cb_ref[0] + cb_ref[1])
    cnt = cntrow[:, 0:1]
    aggr = summed / jnp.maximum(cnt, 1.0)
    xb = x_ref[...]
    out = aggr + jnp.dot(xb, root_ref[...],
                         preferred_element_type=jnp.float32) + bias_ref[...]
    mu = jnp.mean(out, axis=1, keepdims=True)
    var = jnp.mean((out - mu) * (out - mu), axis=1, keepdims=True)
    out = (out - mu) * lax.rsqrt(var + 1e-5) * g_ref[...] + beta_ref[...]
    res = jnp.dot(xb, Wres_ref[...],
                  preferred_element_type=jnp.float32) + bres_ref[...]
    t = out + res
    o_ref[...] = t * jax.nn.sigmoid(t)


def _tc_node(pa, pb, ca, cb, x, root, bias, ln_gamma, ln_beta, Wres, bres):
    grid = (N // BN,)
    return pl.pallas_call(
        _node_body,
        grid=grid,
        in_specs=[
            pl.BlockSpec((2, BN, AUG), lambda i: (0, i, 0)),
            pl.BlockSpec((2, BN, AUG), lambda i: (0, i, 0)),
            pl.BlockSpec((2, BN, CNT_W), lambda i: (0, i, 0)),
            pl.BlockSpec((2, BN, CNT_W), lambda i: (0, i, 0)),
            pl.BlockSpec((BN, IN_C), lambda i: (i, 0)),
            pl.BlockSpec((IN_C, OUT_C), lambda i: (0, 0)),
            pl.BlockSpec((1, OUT_C), lambda i: (0, 0)),
            pl.BlockSpec((1, OUT_C), lambda i: (0, 0)),
            pl.BlockSpec((1, OUT_C), lambda i: (0, 0)),
            pl.BlockSpec((IN_C, OUT_C), lambda i: (0, 0)),
            pl.BlockSpec((1, OUT_C), lambda i: (0, 0)),
        ],
        out_specs=pl.BlockSpec((BN, OUT_C), lambda i: (i, 0)),
        out_shape=jax.ShapeDtypeStruct((N, OUT_C), jnp.float32),
    )(pa, pb, ca, cb, x, root, bias.reshape(1, OUT_C),
      ln_gamma.reshape(1, OUT_C), ln_beta.reshape(1, OUT_C), Wres,
      bres.reshape(1, OUT_C))


def kernel(x, edge_attr, W1, b1, W2, b2, root, bias, ln_gamma, ln_beta,
           Wres, bres, edge_index):
    pad = E_PAD - E
    ei2 = jnp.pad(edge_index, ((0, 0), (0, pad)),
                  constant_values=0).reshape(2, E_PAD // CH, CH)
    slot = jnp.arange(E_PAD).reshape(E_PAD // CH, CH)
    src2 = jnp.where(slot < E, ei2[0], slot % N)
    dst_flat = jnp.where(slot.reshape(E_PAD) < E, ei2[1].reshape(E_PAD), N)
    dst2 = dst_flat.reshape(E_PAD // BE, 4, BE // 4).transpose(
        0, 2, 1).reshape(E_PAD // CH, CH)
    x128 = _tc_pack_x(x)
    zrows = jnp.zeros((N_ACC, AUG), jnp.float32)
    zcnt = jnp.zeros((N_ACC, CNT_W), jnp.float32)
    ones = jnp.ones((CH, CNT_W), jnp.float32)

    msgs = []
    for ci in range(NPIPE):
        xj_c = _sc_gather_chunk(x128, src2, ci)
        m = _tc_edge(edge_attr, xj_c, W1, b1, W2, b2, ci)
        msgs.append(m.reshape(ECH, AUG))
    pa, ca = _sc_scatter(msgs[0:2], dst2, zrows, zcnt, ones, (0, 1))
    pb, cb = _sc_scatter(msgs[2:4], dst2, zrows, zcnt, ones, (2, 3))
    return _tc_node(pa, pb, ca, cb, x, root, bias, ln_gamma, ln_beta,
                    Wres, bres)
